# Initial kernel scaffold; baseline (speedup 1.0000x reference)
import functools

import jax
import jax.numpy as jnp
from jax import lax
from jax.experimental import pallas as pl
from jax.experimental.pallas import tpu as pltpu
from jax.experimental.pallas import tpu_sc as plsc

NUM_CORES = 2
NUM_SUBCORES = 16
NUM_TILES = NUM_CORES * NUM_SUBCORES
CHUNK = 128
HIST_W = 16


def _sc_degree_body(n_acc, dst_hbm, out_hbm, dst_v, ones_v, zero_v, acc):
    cid = lax.axis_index("c")
    sid = lax.axis_index("s")
    w = cid * NUM_SUBCORES + sid
    n_chunks = dst_v.shape[0]
    rows_per_tile = n_acc // NUM_SUBCORES

    def fill(i, carry):
        ones_v[i, :] = jnp.full((16,), 1.0, jnp.float32)
        zero_v[i, :] = jnp.zeros((16,), jnp.float32)
        return carry

    lax.fori_loop(0, CHUNK, fill, 0)

    def zero(j, carry):
        pltpu.sync_copy(
            zero_v, acc.at[pl.ds(sid * rows_per_tile + j * CHUNK, CHUNK)])
        return carry

    lax.fori_loop(0, rows_per_tile // CHUNK, zero, 0)
    pltpu.sync_copy(dst_hbm.at[w], dst_v)
    plsc.subcore_barrier()

    def body(j, carry):
        pltpu.sync_copy(ones_v, acc.at[dst_v.at[j]], add=True)
        return carry

    lax.fori_loop(0, n_chunks, body, 0)
    plsc.subcore_barrier()
    pltpu.sync_copy(
        acc.at[pl.ds(sid * rows_per_tile, rows_per_tile)],
        out_hbm.at[cid, pl.ds(sid * rows_per_tile, rows_per_tile)])


def _make_sc_degree(n_acc, n_chunks):
    mesh = plsc.VectorSubcoreMesh(core_axis_name="c", subcore_axis_name="s")
    return pl.kernel(
        functools.partial(_sc_degree_body, n_acc),
        out_type=jax.ShapeDtypeStruct((NUM_CORES, n_acc, HIST_W), jnp.float32),
        mesh=mesh,
        scratch_types=[
            pltpu.VMEM((n_chunks, CHUNK), jnp.int32),
            pltpu.VMEM((CHUNK, HIST_W), jnp.float32),
            pltpu.VMEM((CHUNK, HIST_W), jnp.float32),
            pltpu.VMEM_SHARED((n_acc, HIST_W), jnp.float32),
        ],
    )


def _sc_gs_body(n_acc, hs_hbm, src_hbm, dst_hbm, out_hbm,
                src_v, dst_v, rows, acc, sem0, sem1):
    cid = lax.axis_index("c")
    sid = lax.axis_index("s")
    w = cid * NUM_SUBCORES + sid
    n_chunks = src_v.shape[0]
    feat = rows.shape[2]
    rows_per_tile = n_acc // NUM_SUBCORES

    def fill(i, carry):
        for c in range(feat // 16):
            rows[0, i, pl.ds(c * 16, 16)] = jnp.zeros((16,), jnp.float32)
        return carry

    lax.fori_loop(0, CHUNK, fill, 0)

    def zero(j, carry):
        pltpu.sync_copy(
            rows.at[0], acc.at[pl.ds(sid * rows_per_tile + j * CHUNK, CHUNK)])
        return carry

    lax.fori_loop(0, rows_per_tile // CHUNK, zero, 0)
    pltpu.sync_copy(src_hbm.at[w], src_v)
    pltpu.sync_copy(dst_hbm.at[w], dst_v)
    plsc.subcore_barrier()

    pltpu.async_copy(hs_hbm.at[src_v.at[0]], rows.at[0], sem0)
    pltpu.async_copy(hs_hbm.at[src_v.at[1]], rows.at[1], sem1)

    def body(t, carry):
        j0 = 2 * t
        j1 = 2 * t + 1
        pltpu.make_async_copy(hs_hbm.at[src_v.at[j0]], rows.at[0], sem0).wait()
        pltpu.sync_copy(rows.at[0], acc.at[dst_v.at[j0]], add=True)

        @pl.when(j0 + 2 < n_chunks)
        def _():
            pltpu.async_copy(hs_hbm.at[src_v.at[j0 + 2]], rows.at[0], sem0)

        pltpu.make_async_copy(hs_hbm.at[src_v.at[j1]], rows.at[1], sem1).wait()
        pltpu.sync_copy(rows.at[1], acc.at[dst_v.at[j1]], add=True)

        @pl.when(j1 + 2 < n_chunks)
        def _():
            pltpu.async_copy(hs_hbm.at[src_v.at[j1 + 2]], rows.at[1], sem1)

        return carry

    lax.fori_loop(0, n_chunks // 2, body, 0)
    plsc.subcore_barrier()
    pltpu.sync_copy(
        acc.at[pl.ds(sid * rows_per_tile, rows_per_tile)],
        out_hbm.at[cid, pl.ds(sid * rows_per_tile, rows_per_tile)])


def _make_sc_gs(n_acc, n_chunks, feat):
    mesh = plsc.VectorSubcoreMesh(core_axis_name="c", subcore_axis_name="s")
    return pl.kernel(
        functools.partial(_sc_gs_body, n_acc),
        out_type=jax.ShapeDtypeStruct((NUM_CORES, n_acc, feat), jnp.float32),
        mesh=mesh,
        scratch_types=[
            pltpu.VMEM((n_chunks, CHUNK), jnp.int32),
            pltpu.VMEM((n_chunks, CHUNK), jnp.int32),
            pltpu.VMEM((2, CHUNK, feat), jnp.float32),
            pltpu.VMEM_SHARED((n_acc, feat), jnp.float32),
            pltpu.SemaphoreType.DMA,
            pltpu.SemaphoreType.DMA,
        ],
    )


def _dinv_body(hist_ref, dinv_ref):
    deg = 1.0 + hist_ref[0] + hist_ref[1]
    dinv_ref[...] = lax.rsqrt(deg)


def _mm1_body(x_ref, w_ref, dinv_ref, h_ref, hs_ref):
    h = jnp.dot(x_ref[...], w_ref[...], preferred_element_type=jnp.float32)
    h_ref[...] = h
    hs_ref[...] = h * dinv_ref[...]


def _mid_body(s_ref, h1_ref, dinv_ref, b1_ref, w2_ref, h2_ref, hs2_ref):
    dinv = dinv_ref[...]
    s = s_ref[0] + s_ref[1]
    h = jnp.maximum(dinv * s + dinv * dinv * h1_ref[...] + b1_ref[...], 0.0)
    h2 = jnp.dot(h, w2_ref[...], preferred_element_type=jnp.float32)
    h2_ref[...] = h2
    hs2_ref[...] = h2 * dinv


def _final_body(s_ref, h2_ref, dinv_ref, b2_ref, o_ref):
    dinv = dinv_ref[...]
    s = s_ref[0] + s_ref[1]
    o = dinv * s + dinv * dinv * h2_ref[...] + b2_ref[...]
    m = jnp.max(o, axis=1, keepdims=True)
    z = o - m
    o_ref[...] = z - jnp.log(jnp.sum(jnp.exp(z), axis=1, keepdims=True))


def kernel(x, edge_index, W1, b1, W2, b2):
    n = x.shape[0]
    e = edge_index.shape[1]
    f_in = x.shape[1]
    hid = W1.shape[1]
    cls = W2.shape[1]

    n_chunks = -(-e // (NUM_TILES * CHUNK))
    n_chunks += n_chunks % 2
    slots = NUM_TILES * n_chunks * CHUNK
    pad = slots - e
    src = jnp.concatenate(
        [edge_index[0], jnp.zeros((pad,), jnp.int32)]).reshape(
            NUM_TILES, n_chunks, CHUNK)
    dst = jnp.concatenate(
        [edge_index[1], jnp.full((pad,), n, jnp.int32)]).reshape(
            NUM_TILES, n_chunks, CHUNK)

    n_acc = NUM_SUBCORES * CHUNK * (-(-(n + 1) // (NUM_SUBCORES * CHUNK)))

    hist = _make_sc_degree(n_acc, n_chunks)(dst)

    dinv_full = pl.pallas_call(
        _dinv_body,
        out_shape=jax.ShapeDtypeStruct((n_acc, HIST_W), jnp.float32),
    )(hist)
    dinv = dinv_full[:n, 0:1]

    row_blk = 2000
    grid = (n // row_blk,)

    H1, Hs1 = pl.pallas_call(
        _mm1_body,
        grid=grid,
        in_specs=[
            pl.BlockSpec((row_blk, f_in), lambda i: (i, 0)),
            pl.BlockSpec((f_in, hid), lambda i: (0, 0)),
            pl.BlockSpec((row_blk, 1), lambda i: (i, 0)),
        ],
        out_specs=[
            pl.BlockSpec((row_blk, hid), lambda i: (i, 0)),
            pl.BlockSpec((row_blk, hid), lambda i: (i, 0)),
        ],
        out_shape=[jax.ShapeDtypeStruct((n, hid), jnp.float32)] * 2,
    )(x, W1, dinv)

    S1 = _make_sc_gs(n_acc, n_chunks, hid)(Hs1, src, dst)

    H2, Hs2 = pl.pallas_call(
        _mid_body,
        grid=grid,
        in_specs=[
            pl.BlockSpec((NUM_CORES, row_blk, hid), lambda i: (0, i, 0)),
            pl.BlockSpec((row_blk, hid), lambda i: (i, 0)),
            pl.BlockSpec((row_blk, 1), lambda i: (i, 0)),
            pl.BlockSpec((1, hid), lambda i: (0, 0)),
            pl.BlockSpec((hid, cls), lambda i: (0, 0)),
        ],
        out_specs=[
            pl.BlockSpec((row_blk, cls), lambda i: (i, 0)),
            pl.BlockSpec((row_blk, cls), lambda i: (i, 0)),
        ],
        out_shape=[jax.ShapeDtypeStruct((n, cls), jnp.float32)] * 2,
    )(S1[:, :n], H1, dinv, b1.reshape(1, hid), W2)

    S2 = _make_sc_gs(n_acc, n_chunks, cls)(Hs2, src, dst)

    out = pl.pallas_call(
        _final_body,
        grid=grid,
        in_specs=[
            pl.BlockSpec((NUM_CORES, row_blk, cls), lambda i: (0, i, 0)),
            pl.BlockSpec((row_blk, cls), lambda i: (i, 0)),
            pl.BlockSpec((row_blk, 1), lambda i: (i, 0)),
            pl.BlockSpec((1, cls), lambda i: (0, 0)),
        ],
        out_specs=pl.BlockSpec((row_blk, cls), lambda i: (i, 0)),
        out_shape=jax.ShapeDtypeStruct((n, cls), jnp.float32),
    )(S2[:, :n], H2, dinv, b2.reshape(1, cls))

    return out


# speedup vs baseline: 11.3479x; 11.3479x over previous
import functools

import jax
import jax.numpy as jnp
from jax import lax
from jax.experimental import pallas as pl
from jax.experimental.pallas import tpu as pltpu
from jax.experimental.pallas import tpu_sc as plsc

NUM_CORES = 2
NUM_SUBCORES = 16
NUM_TILES = NUM_CORES * NUM_SUBCORES
CHUNK = 128
IDXB = 16
HIST_W = 16


def _sc_degree_body(n_acc, dst_hbm, out_hbm, dst_v, ones_v, zero_v, acc):
    cid = lax.axis_index("c")
    sid = lax.axis_index("s")
    w = cid * NUM_SUBCORES + sid
    n_chunks = dst_v.shape[0]
    rows_per_tile = n_acc // NUM_SUBCORES

    def fill(i, carry):
        ones_v[i, :] = jnp.full((16,), 1.0, jnp.float32)
        zero_v[i, :] = jnp.zeros((16,), jnp.float32)
        return carry

    lax.fori_loop(0, CHUNK, fill, 0)

    def zero(j, carry):
        pltpu.sync_copy(
            zero_v, acc.at[pl.ds(sid * rows_per_tile + j * CHUNK, CHUNK)])
        return carry

    lax.fori_loop(0, rows_per_tile // CHUNK, zero, 0)
    pltpu.sync_copy(dst_hbm.at[w], dst_v)
    plsc.subcore_barrier()

    def body(j, carry):
        pltpu.sync_copy(ones_v, acc.at[dst_v.at[j]], add=True)
        return carry

    lax.fori_loop(0, n_chunks, body, 0)
    plsc.subcore_barrier()
    pltpu.sync_copy(
        acc.at[pl.ds(sid * rows_per_tile, rows_per_tile)],
        out_hbm.at[cid, pl.ds(sid * rows_per_tile, rows_per_tile)])


def _make_sc_degree(n_acc, n_chunks):
    mesh = plsc.VectorSubcoreMesh(core_axis_name="c", subcore_axis_name="s")
    return pl.kernel(
        functools.partial(_sc_degree_body, n_acc),
        out_type=jax.ShapeDtypeStruct((NUM_CORES, n_acc, HIST_W), jnp.float32),
        mesh=mesh,
        scratch_types=[
            pltpu.VMEM((n_chunks, CHUNK), jnp.int32),
            pltpu.VMEM((CHUNK, HIST_W), jnp.float32),
            pltpu.VMEM((CHUNK, HIST_W), jnp.float32),
            pltpu.VMEM_SHARED((n_acc, HIST_W), jnp.float32),
        ],
    )


def _sc_gs_body(n_acc, hs_hbm, src_hbm, dst_hbm, out_hbm,
                src_v, dst_v, rows, acc, sem0, sem1):
    cid = lax.axis_index("c")
    sid = lax.axis_index("s")
    w = cid * NUM_SUBCORES + sid
    n_chunks = src_hbm.shape[1]
    feat = rows.shape[2]
    rows_per_tile = n_acc // NUM_SUBCORES

    def fill(i, carry):
        for c in range(feat // 16):
            rows[0, i, pl.ds(c * 16, 16)] = jnp.zeros((16,), jnp.float32)
        return carry

    lax.fori_loop(0, CHUNK, fill, 0)

    def zero(j, carry):
        pltpu.sync_copy(
            rows.at[0], acc.at[pl.ds(sid * rows_per_tile + j * CHUNK, CHUNK)])
        return carry

    lax.fori_loop(0, rows_per_tile // CHUNK, zero, 0)
    plsc.subcore_barrier()

    def group(g, carry):
        pltpu.sync_copy(src_hbm.at[w, pl.ds(g * IDXB, IDXB)], src_v)
        pltpu.sync_copy(dst_hbm.at[w, pl.ds(g * IDXB, IDXB)], dst_v)
        pltpu.async_copy(hs_hbm.at[src_v.at[0]], rows.at[0], sem0)
        pltpu.async_copy(hs_hbm.at[src_v.at[1]], rows.at[1], sem1)

        def body(t, carry2):
            j0 = 2 * t
            j1 = 2 * t + 1
            pltpu.make_async_copy(
                hs_hbm.at[src_v.at[j0]], rows.at[0], sem0).wait()
            pltpu.sync_copy(rows.at[0], acc.at[dst_v.at[j0]], add=True)

            @pl.when(j0 + 2 < IDXB)
            def _():
                pltpu.async_copy(hs_hbm.at[src_v.at[j0 + 2]], rows.at[0], sem0)

            pltpu.make_async_copy(
                hs_hbm.at[src_v.at[j1]], rows.at[1], sem1).wait()
            pltpu.sync_copy(rows.at[1], acc.at[dst_v.at[j1]], add=True)

            @pl.when(j1 + 2 < IDXB)
            def _():
                pltpu.async_copy(hs_hbm.at[src_v.at[j1 + 2]], rows.at[1], sem1)

            return carry2

        lax.fori_loop(0, IDXB // 2, body, 0)
        return carry

    lax.fori_loop(0, n_chunks // IDXB, group, 0)
    plsc.subcore_barrier()
    pltpu.sync_copy(
        acc.at[pl.ds(sid * rows_per_tile, rows_per_tile)],
        out_hbm.at[cid, pl.ds(sid * rows_per_tile, rows_per_tile)])


def _make_sc_gs(n_acc, n_chunks, feat):
    mesh = plsc.VectorSubcoreMesh(core_axis_name="c", subcore_axis_name="s")
    return pl.kernel(
        functools.partial(_sc_gs_body, n_acc),
        out_type=jax.ShapeDtypeStruct((NUM_CORES, n_acc, feat), jnp.float32),
        mesh=mesh,
        compiler_params=pltpu.CompilerParams(use_tc_tiling_on_sc=False),
        scratch_types=[
            pltpu.VMEM((IDXB, CHUNK), jnp.int32),
            pltpu.VMEM((IDXB, CHUNK), jnp.int32),
            pltpu.VMEM((2, CHUNK, feat), jnp.float32),
            pltpu.VMEM_SHARED((n_acc, feat), jnp.float32),
            pltpu.SemaphoreType.DMA,
            pltpu.SemaphoreType.DMA,
        ],
    )


def _dinv_body(hist_ref, dinv_ref):
    deg = 1.0 + hist_ref[0] + hist_ref[1]
    dinv_ref[...] = lax.rsqrt(deg)


def _mm1_body(x_ref, w_ref, dinv_ref, h_ref, hs_ref):
    h = jnp.dot(x_ref[...], w_ref[...], preferred_element_type=jnp.float32)
    h_ref[...] = h
    hs_ref[...] = h * dinv_ref[...]


def _mid_body(s_ref, h1_ref, dinv_ref, b1_ref, w2_ref, h2_ref, hs2_ref):
    dinv = dinv_ref[...]
    s = s_ref[0] + s_ref[1]
    h = jnp.maximum(dinv * s + dinv * dinv * h1_ref[...] + b1_ref[...], 0.0)
    h2 = jnp.dot(h, w2_ref[...], preferred_element_type=jnp.float32)
    h2_ref[...] = h2
    hs2_ref[...] = h2 * dinv


def _final_body(s_ref, h2_ref, dinv_ref, b2_ref, o_ref):
    dinv = dinv_ref[...]
    s = s_ref[0] + s_ref[1]
    o = dinv * s + dinv * dinv * h2_ref[...] + b2_ref[...]
    m = jnp.max(o, axis=1, keepdims=True)
    z = o - m
    o_ref[...] = z - jnp.log(jnp.sum(jnp.exp(z), axis=1, keepdims=True))


def kernel(x, edge_index, W1, b1, W2, b2):
    n = x.shape[0]
    e = edge_index.shape[1]
    f_in = x.shape[1]
    hid = W1.shape[1]
    cls = W2.shape[1]

    n_chunks = IDXB * (-(-e // (NUM_TILES * CHUNK * IDXB)))
    slots = NUM_TILES * n_chunks * CHUNK
    pad = slots - e
    src = jnp.concatenate(
        [edge_index[0], jnp.zeros((pad,), jnp.int32)]).reshape(
            NUM_TILES, n_chunks, CHUNK)
    dst = jnp.concatenate(
        [edge_index[1], jnp.full((pad,), n, jnp.int32)]).reshape(
            NUM_TILES, n_chunks, CHUNK)

    n_acc = NUM_SUBCORES * CHUNK * (-(-(n + 1) // (NUM_SUBCORES * CHUNK)))

    hist = _make_sc_degree(n_acc, n_chunks)(dst)

    dinv_full = pl.pallas_call(
        _dinv_body,
        out_shape=jax.ShapeDtypeStruct((n_acc, HIST_W), jnp.float32),
    )(hist)
    dinv = dinv_full[:n, 0:1]

    row_blk = 2000
    grid = (n // row_blk,)

    H1, Hs1 = pl.pallas_call(
        _mm1_body,
        grid=grid,
        in_specs=[
            pl.BlockSpec((row_blk, f_in), lambda i: (i, 0)),
            pl.BlockSpec((f_in, hid), lambda i: (0, 0)),
            pl.BlockSpec((row_blk, 1), lambda i: (i, 0)),
        ],
        out_specs=[
            pl.BlockSpec((row_blk, hid), lambda i: (i, 0)),
            pl.BlockSpec((row_blk, hid), lambda i: (i, 0)),
        ],
        out_shape=[jax.ShapeDtypeStruct((n, hid), jnp.float32)] * 2,
    )(x, W1, dinv)

    S1 = _make_sc_gs(n_acc, n_chunks, hid)(Hs1, src, dst)

    H2, Hs2 = pl.pallas_call(
        _mid_body,
        grid=grid,
        in_specs=[
            pl.BlockSpec((NUM_CORES, row_blk, hid), lambda i: (0, i, 0)),
            pl.BlockSpec((row_blk, hid), lambda i: (i, 0)),
            pl.BlockSpec((row_blk, 1), lambda i: (i, 0)),
            pl.BlockSpec((1, hid), lambda i: (0, 0)),
            pl.BlockSpec((hid, cls), lambda i: (0, 0)),
        ],
        out_specs=[
            pl.BlockSpec((row_blk, cls), lambda i: (i, 0)),
            pl.BlockSpec((row_blk, cls), lambda i: (i, 0)),
        ],
        out_shape=[jax.ShapeDtypeStruct((n, cls), jnp.float32)] * 2,
    )(S1[:, :n], H1, dinv, b1.reshape(1, hid), W2)

    S2 = _make_sc_gs(n_acc, n_chunks, cls)(Hs2, src, dst)

    out = pl.pallas_call(
        _final_body,
        grid=grid,
        in_specs=[
            pl.BlockSpec((NUM_CORES, row_blk, cls), lambda i: (0, i, 0)),
            pl.BlockSpec((row_blk, cls), lambda i: (i, 0)),
            pl.BlockSpec((row_blk, 1), lambda i: (i, 0)),
            pl.BlockSpec((1, cls), lambda i: (0, 0)),
        ],
        out_specs=pl.BlockSpec((row_blk, cls), lambda i: (i, 0)),
        out_shape=jax.ShapeDtypeStruct((n, cls), jnp.float32),
    )(S2[:, :n], H2, dinv, b2.reshape(1, cls))

    return out


# speedup vs baseline: 23.5096x; 2.0717x over previous
import functools

import jax
import jax.numpy as jnp
from jax import lax
from jax.experimental import pallas as pl
from jax.experimental.pallas import tpu as pltpu
from jax.experimental.pallas import tpu_sc as plsc

NUM_CORES = 2
NUM_SUBCORES = 16
NUM_TILES = NUM_CORES * NUM_SUBCORES
CHUNK = 128
IDXB = 16
HIST_W = 16


def _sc_degree_body(n_acc, dst_hbm, out_hbm, dst_v, ones_v, zero_v, acc):
    cid = lax.axis_index("c")
    sid = lax.axis_index("s")
    w = cid * NUM_SUBCORES + sid
    n_chunks = dst_v.shape[0]
    rows_per_tile = n_acc // NUM_SUBCORES

    def fill(i, carry):
        ones_v[i, :] = jnp.full((16,), 1.0, jnp.float32)
        zero_v[i, :] = jnp.zeros((16,), jnp.float32)
        return carry

    lax.fori_loop(0, CHUNK, fill, 0)

    def zero(j, carry):
        pltpu.sync_copy(
            zero_v, acc.at[pl.ds(sid * rows_per_tile + j * CHUNK, CHUNK)])
        return carry

    lax.fori_loop(0, rows_per_tile // CHUNK, zero, 0)
    pltpu.sync_copy(dst_hbm.at[w], dst_v)
    plsc.subcore_barrier()

    def body(j, carry):
        pltpu.sync_copy(ones_v, acc.at[dst_v.at[j]], add=True)
        return carry

    lax.fori_loop(0, n_chunks, body, 0)
    plsc.subcore_barrier()
    pltpu.sync_copy(
        acc.at[pl.ds(sid * rows_per_tile, rows_per_tile)],
        out_hbm.at[cid, pl.ds(sid * rows_per_tile, rows_per_tile)])


def _make_sc_degree(n_acc, n_chunks):
    mesh = plsc.VectorSubcoreMesh(core_axis_name="c", subcore_axis_name="s")
    return pl.kernel(
        functools.partial(_sc_degree_body, n_acc),
        out_type=jax.ShapeDtypeStruct((NUM_CORES, n_acc, HIST_W), jnp.float32),
        mesh=mesh,
        scratch_types=[
            pltpu.VMEM((n_chunks, CHUNK), jnp.int32),
            pltpu.VMEM((CHUNK, HIST_W), jnp.float32),
            pltpu.VMEM((CHUNK, HIST_W), jnp.float32),
            pltpu.VMEM_SHARED((n_acc, HIST_W), jnp.float32),
        ],
    )


def _sc_gs_body(n_acc, col_split, use_table, *refs):
    if col_split:
        hs_a, hs_b, src_hbm, dst_hbm, out_hbm = refs[:5]
    else:
        hs_a, src_hbm, dst_hbm, out_hbm = refs[:4]
        refs = (None,) + refs
    src_v, dst_v, rows, table, acc, sem0, sem1 = refs[5:]
    cid = lax.axis_index("c")
    sid = lax.axis_index("s")
    w = sid if col_split else cid * NUM_SUBCORES + sid
    n_chunks = src_hbm.shape[1]
    feat = rows.shape[2]
    rows_per_tile = n_acc // NUM_SUBCORES
    n_table = table.shape[0]
    stage_rows = n_table // NUM_SUBCORES

    if use_table:
        if col_split:
            @pl.when(cid == 0)
            def _():
                pltpu.sync_copy(hs_a.at[pl.ds(sid * stage_rows, stage_rows)],
                                table.at[pl.ds(sid * stage_rows, stage_rows)])

            @pl.when(cid == 1)
            def _():
                pltpu.sync_copy(hs_b.at[pl.ds(sid * stage_rows, stage_rows)],
                                table.at[pl.ds(sid * stage_rows, stage_rows)])
        else:
            pltpu.sync_copy(hs_a.at[pl.ds(sid * stage_rows, stage_rows)],
                            table.at[pl.ds(sid * stage_rows, stage_rows)])
        gather_src = table
    else:
        gather_src = hs_a

    def fill(i, carry):
        for c in range(feat // 16):
            rows[0, i, pl.ds(c * 16, 16)] = jnp.zeros((16,), jnp.float32)
        return carry

    lax.fori_loop(0, CHUNK, fill, 0)

    def zero(j, carry):
        pltpu.sync_copy(
            rows.at[0], acc.at[pl.ds(sid * rows_per_tile + j * CHUNK, CHUNK)])
        return carry

    lax.fori_loop(0, rows_per_tile // CHUNK, zero, 0)
    plsc.subcore_barrier()

    def group(g, carry):
        pltpu.sync_copy(src_hbm.at[w, pl.ds(g * IDXB, IDXB)], src_v)
        pltpu.sync_copy(dst_hbm.at[w, pl.ds(g * IDXB, IDXB)], dst_v)
        pltpu.async_copy(gather_src.at[src_v.at[0]], rows.at[0], sem0)
        pltpu.async_copy(gather_src.at[src_v.at[1]], rows.at[1], sem1)

        def body(t, carry2):
            j0 = 2 * t
            j1 = 2 * t + 1
            pltpu.make_async_copy(
                gather_src.at[src_v.at[j0]], rows.at[0], sem0).wait()
            pltpu.sync_copy(rows.at[0], acc.at[dst_v.at[j0]], add=True)

            @pl.when(j0 + 2 < IDXB)
            def _():
                pltpu.async_copy(gather_src.at[src_v.at[j0 + 2]], rows.at[0], sem0)

            pltpu.make_async_copy(
                gather_src.at[src_v.at[j1]], rows.at[1], sem1).wait()
            pltpu.sync_copy(rows.at[1], acc.at[dst_v.at[j1]], add=True)

            @pl.when(j1 + 2 < IDXB)
            def _():
                pltpu.async_copy(gather_src.at[src_v.at[j1 + 2]], rows.at[1], sem1)

            return carry2

        lax.fori_loop(0, IDXB // 2, body, 0)
        return carry

    lax.fori_loop(0, n_chunks // IDXB, group, 0)
    plsc.subcore_barrier()
    pltpu.sync_copy(
        acc.at[pl.ds(sid * rows_per_tile, rows_per_tile)],
        out_hbm.at[cid, pl.ds(sid * rows_per_tile, rows_per_tile)])


def _make_sc_gs(n_acc, n_table, feat, col_split, use_table=True):
    mesh = plsc.VectorSubcoreMesh(core_axis_name="c", subcore_axis_name="s")
    return pl.kernel(
        functools.partial(_sc_gs_body, n_acc, col_split, use_table),
        out_type=jax.ShapeDtypeStruct((NUM_CORES, n_acc, feat), jnp.float32),
        mesh=mesh,
        compiler_params=pltpu.CompilerParams(use_tc_tiling_on_sc=False),
        scratch_types=[
            pltpu.VMEM((IDXB, CHUNK), jnp.int32),
            pltpu.VMEM((IDXB, CHUNK), jnp.int32),
            pltpu.VMEM((2, CHUNK, feat), jnp.float32),
            pltpu.VMEM_SHARED((n_table, feat), jnp.float32),
            pltpu.VMEM_SHARED((n_acc, feat), jnp.float32),
            pltpu.SemaphoreType.DMA,
            pltpu.SemaphoreType.DMA,
        ],
    )


def _dinv_body(hist_ref, dinv_ref):
    deg = 1.0 + hist_ref[0] + hist_ref[1]
    dinv_ref[...] = lax.rsqrt(deg)


def _mm1_body(x_ref, w_ref, dinv_ref, h_ref, hs_ref):
    h = jnp.dot(x_ref[...], w_ref[...], preferred_element_type=jnp.float32)
    h_ref[...] = h
    hs_ref[...] = h * dinv_ref[...]


def _mid_body(s_ref, h1_ref, dinv_ref, b1_ref, w2_ref, h2_ref, hs2_ref):
    dinv = dinv_ref[...]
    d2 = dinv * dinv
    width = s_ref.shape[2]
    if width == h1_ref.shape[1]:
        s = s_ref[0] + s_ref[1]
        h = jnp.maximum(dinv * s + d2 * h1_ref[...] + b1_ref[...], 0.0)
        h2 = jnp.dot(h, w2_ref[...], preferred_element_type=jnp.float32)
    else:
        half = width
        ha = jnp.maximum(
            dinv * s_ref[0] + d2 * h1_ref[:, :half] + b1_ref[:, :half], 0.0)
        hb = jnp.maximum(
            dinv * s_ref[1] + d2 * h1_ref[:, half:] + b1_ref[:, half:], 0.0)
        h2 = (jnp.dot(ha, w2_ref[:half], preferred_element_type=jnp.float32)
              + jnp.dot(hb, w2_ref[half:], preferred_element_type=jnp.float32))
    h2_ref[...] = h2
    hs2_ref[...] = h2 * dinv


def _final_body(s_ref, h2_ref, dinv_ref, b2_ref, o_ref):
    dinv = dinv_ref[...]
    s = s_ref[0] + s_ref[1]
    o = dinv * s + dinv * dinv * h2_ref[...] + b2_ref[...]
    m = jnp.max(o, axis=1, keepdims=True)
    z = o - m
    o_ref[...] = z - jnp.log(jnp.sum(jnp.exp(z), axis=1, keepdims=True))


def kernel(x, edge_index, W1, b1, W2, b2):
    n = x.shape[0]
    e = edge_index.shape[1]
    f_in = x.shape[1]
    hid = W1.shape[1]
    cls = W2.shape[1]

    def edge_layout(n_ways):
        n_chunks = IDXB * (-(-e // (n_ways * CHUNK * IDXB)))
        pad = n_ways * n_chunks * CHUNK - e
        s = jnp.concatenate(
            [edge_index[0], jnp.zeros((pad,), jnp.int32)]).reshape(
                n_ways, n_chunks, CHUNK)
        d = jnp.concatenate(
            [edge_index[1], jnp.full((pad,), n, jnp.int32)]).reshape(
                n_ways, n_chunks, CHUNK)
        return s, d

    src32, dst32 = edge_layout(NUM_TILES)
    src16, dst16 = edge_layout(NUM_SUBCORES)

    n_acc = NUM_SUBCORES * CHUNK * (-(-(n + 1) // (NUM_SUBCORES * CHUNK)))

    hist = _make_sc_degree(n_acc, dst32.shape[1])(dst32)

    dinv_full = pl.pallas_call(
        _dinv_body,
        out_shape=jax.ShapeDtypeStruct((n_acc, HIST_W), jnp.float32),
    )(hist)
    dinv = dinv_full[:n, 0:1]

    row_blk = 2000
    grid = (n // row_blk,)

    H1, Hs1 = pl.pallas_call(
        _mm1_body,
        grid=grid,
        in_specs=[
            pl.BlockSpec((row_blk, f_in), lambda i: (i, 0)),
            pl.BlockSpec((f_in, hid), lambda i: (0, 0)),
            pl.BlockSpec((row_blk, 1), lambda i: (i, 0)),
        ],
        out_specs=[
            pl.BlockSpec((row_blk, hid), lambda i: (i, 0)),
            pl.BlockSpec((row_blk, hid), lambda i: (i, 0)),
        ],
        out_shape=[jax.ShapeDtypeStruct((n, hid), jnp.float32)] * 2,
    )(x, W1, dinv)

    s1_width = hid // 2
    if s1_width == hid:
        S1 = _make_sc_gs(n_acc, NUM_SUBCORES, hid, False,
                         use_table=False)(Hs1, src32, dst32)
    else:
        S1 = _make_sc_gs(n_acc, n, s1_width, True)(
            Hs1[:, :s1_width], Hs1[:, s1_width:], src16, dst16)

    H2, Hs2 = pl.pallas_call(
        _mid_body,
        grid=grid,
        in_specs=[
            pl.BlockSpec((NUM_CORES, row_blk, s1_width), lambda i: (0, i, 0)),
            pl.BlockSpec((row_blk, hid), lambda i: (i, 0)),
            pl.BlockSpec((row_blk, 1), lambda i: (i, 0)),
            pl.BlockSpec((1, hid), lambda i: (0, 0)),
            pl.BlockSpec((hid, cls), lambda i: (0, 0)),
        ],
        out_specs=[
            pl.BlockSpec((row_blk, cls), lambda i: (i, 0)),
            pl.BlockSpec((row_blk, cls), lambda i: (i, 0)),
        ],
        out_shape=[jax.ShapeDtypeStruct((n, cls), jnp.float32)] * 2,
    )(S1[:, :n], H1, dinv, b1.reshape(1, hid), W2)

    S2 = _make_sc_gs(n_acc, n, cls, False)(Hs2, src32, dst32)

    out = pl.pallas_call(
        _final_body,
        grid=grid,
        in_specs=[
            pl.BlockSpec((NUM_CORES, row_blk, cls), lambda i: (0, i, 0)),
            pl.BlockSpec((row_blk, cls), lambda i: (i, 0)),
            pl.BlockSpec((row_blk, 1), lambda i: (i, 0)),
            pl.BlockSpec((1, cls), lambda i: (0, 0)),
        ],
        out_specs=pl.BlockSpec((row_blk, cls), lambda i: (i, 0)),
        out_shape=jax.ShapeDtypeStruct((n, cls), jnp.float32),
    )(S2[:, :n], H2, dinv, b2.reshape(1, cls))

    return out


# speedup vs baseline: 25.8760x; 1.1007x over previous
import functools

import jax
import jax.numpy as jnp
from jax import lax
from jax.experimental import pallas as pl
from jax.experimental.pallas import tpu as pltpu
from jax.experimental.pallas import tpu_sc as plsc

NUM_CORES = 2
NUM_SUBCORES = 16
NUM_TILES = NUM_CORES * NUM_SUBCORES
CHUNK = 128
NBUF = 4
HIST_W = 16


def _sc_degree_body(n_acc, dst_hbm, out_hbm, dst_v, ones_v, zero_v, acc):
    cid = lax.axis_index("c")
    sid = lax.axis_index("s")
    w = cid * NUM_SUBCORES + sid
    n_chunks = dst_v.shape[0]
    rows_per_tile = n_acc // NUM_SUBCORES

    def fill(i, carry):
        ones_v[i, :] = jnp.full((16,), 1.0, jnp.float32)
        zero_v[i, :] = jnp.zeros((16,), jnp.float32)
        return carry

    lax.fori_loop(0, CHUNK, fill, 0)

    def zero(j, carry):
        pltpu.sync_copy(
            zero_v, acc.at[pl.ds(sid * rows_per_tile + j * CHUNK, CHUNK)])
        return carry

    lax.fori_loop(0, rows_per_tile // CHUNK, zero, 0)
    pltpu.sync_copy(dst_hbm.at[w], dst_v)
    plsc.subcore_barrier()

    def body(j, carry):
        pltpu.sync_copy(ones_v, acc.at[dst_v.at[j]], add=True)
        return carry

    lax.fori_loop(0, n_chunks, body, 0)
    plsc.subcore_barrier()
    pltpu.sync_copy(
        acc.at[pl.ds(sid * rows_per_tile, rows_per_tile)],
        out_hbm.at[cid, pl.ds(sid * rows_per_tile, rows_per_tile)])


def _make_sc_degree(n_acc, n_chunks):
    mesh = plsc.VectorSubcoreMesh(core_axis_name="c", subcore_axis_name="s")
    return pl.kernel(
        functools.partial(_sc_degree_body, n_acc),
        out_type=jax.ShapeDtypeStruct((NUM_CORES, n_acc, HIST_W), jnp.float32),
        mesh=mesh,
        scratch_types=[
            pltpu.VMEM((n_chunks, CHUNK), jnp.int32),
            pltpu.VMEM((CHUNK, HIST_W), jnp.float32),
            pltpu.VMEM((CHUNK, HIST_W), jnp.float32),
            pltpu.VMEM_SHARED((n_acc, HIST_W), jnp.float32),
        ],
    )


def _sc_gs_body(n_acc, col_split, idxb, *refs):
    if col_split:
        hs_a, hs_b, src_hbm, dst_hbm, out_hbm = refs[:5]
        rest = refs[5:]
    else:
        hs_a, src_hbm, dst_hbm, out_hbm = refs[:4]
        rest = refs[4:]
    src_v, dst_v, rows = rest[:3]
    table, acc = rest[3:5]
    semg = rest[5:5 + NBUF]
    sems = rest[5 + NBUF:5 + 2 * NBUF]
    cid = lax.axis_index("c")
    sid = lax.axis_index("s")
    w = sid if col_split else cid * NUM_SUBCORES + sid
    n_chunks = src_hbm.shape[1]
    feat = rows.shape[2]
    rows_per_tile = n_acc // NUM_SUBCORES
    stage_rows = table.shape[0] // NUM_SUBCORES

    if col_split:
        @pl.when(cid == 0)
        def _():
            pltpu.sync_copy(hs_a.at[pl.ds(sid * stage_rows, stage_rows)],
                            table.at[pl.ds(sid * stage_rows, stage_rows)])

        @pl.when(cid == 1)
        def _():
            pltpu.sync_copy(hs_b.at[pl.ds(sid * stage_rows, stage_rows)],
                            table.at[pl.ds(sid * stage_rows, stage_rows)])
    else:
        pltpu.sync_copy(hs_a.at[pl.ds(sid * stage_rows, stage_rows)],
                        table.at[pl.ds(sid * stage_rows, stage_rows)])

    def fill(i, carry):
        for c in range(feat // 16):
            rows[0, i, pl.ds(c * 16, 16)] = jnp.zeros((16,), jnp.float32)
        return carry

    lax.fori_loop(0, CHUNK, fill, 0)

    def zero(j, carry):
        pltpu.sync_copy(
            rows.at[0], acc.at[pl.ds(sid * rows_per_tile + j * CHUNK, CHUNK)])
        return carry

    lax.fori_loop(0, rows_per_tile // CHUNK, zero, 0)
    plsc.subcore_barrier()

    def group(g, carry):
        pltpu.sync_copy(src_hbm.at[w, pl.ds(g * idxb, idxb)], src_v)
        pltpu.sync_copy(dst_hbm.at[w, pl.ds(g * idxb, idxb)], dst_v)
        for b in range(NBUF):
            pltpu.async_copy(table.at[src_v.at[b]], rows.at[b], semg[b])

        def body(t, carry2):
            for b in range(NBUF):
                j = NBUF * t + b
                pltpu.make_async_copy(
                    table.at[src_v.at[j]], rows.at[b], semg[b]).wait()
                pltpu.sync_copy(rows.at[b], acc.at[dst_v.at[j]], add=True)

                @pl.when(j + NBUF < idxb)
                def _():
                    pltpu.async_copy(
                        table.at[src_v.at[j + NBUF]], rows.at[b], semg[b])

            return carry2

        lax.fori_loop(0, idxb // NBUF, body, 0)
        return carry

    lax.fori_loop(0, n_chunks // idxb, group, 0)
    plsc.subcore_barrier()
    pltpu.sync_copy(
        acc.at[pl.ds(sid * rows_per_tile, rows_per_tile)],
        out_hbm.at[cid, pl.ds(sid * rows_per_tile, rows_per_tile)])


def _make_sc_gs(n_acc, n_table, feat, col_split, idxb):
    mesh = plsc.VectorSubcoreMesh(core_axis_name="c", subcore_axis_name="s")
    return pl.kernel(
        functools.partial(_sc_gs_body, n_acc, col_split, idxb),
        out_type=jax.ShapeDtypeStruct((NUM_CORES, n_acc, feat), jnp.float32),
        mesh=mesh,
        compiler_params=pltpu.CompilerParams(use_tc_tiling_on_sc=False),
        scratch_types=[
            pltpu.VMEM((idxb, CHUNK), jnp.int32),
            pltpu.VMEM((idxb, CHUNK), jnp.int32),
            pltpu.VMEM((NBUF, CHUNK, feat), jnp.float32),
            pltpu.VMEM_SHARED((n_table, feat), jnp.float32),
            pltpu.VMEM_SHARED((n_acc, feat), jnp.float32),
        ] + [pltpu.SemaphoreType.DMA] * (2 * NBUF),
    )


def _mm1_body(x_ref, w_ref, h_ref):
    h_ref[...] = jnp.dot(x_ref[...], w_ref[...],
                         preferred_element_type=jnp.float32)


def _scale_body(hist_ref, h1_ref, dinv_ref, hsa_ref, hsb_ref):
    half = hsa_ref.shape[1]
    deg = 1.0 + hist_ref[0, :, 0:1] + hist_ref[1, :, 0:1]
    dinv = lax.rsqrt(deg)
    dinv_ref[...] = dinv
    hs = h1_ref[...] * dinv
    hsa_ref[...] = hs[:, :half]
    hsb_ref[...] = hs[:, half:]


def _mid_body(sa_ref, sb_ref, h1_ref, dinv_ref, b1_ref, w2_ref,
              h2_ref, hs2_ref):
    dinv = dinv_ref[...]
    d2 = dinv * dinv
    half = sa_ref.shape[2]
    ha = jnp.maximum(
        dinv * sa_ref[0] + d2 * h1_ref[:, :half] + b1_ref[:, :half], 0.0)
    hb = jnp.maximum(
        dinv * sb_ref[0] + d2 * h1_ref[:, half:] + b1_ref[:, half:], 0.0)
    h2 = (jnp.dot(ha, w2_ref[:half], preferred_element_type=jnp.float32)
          + jnp.dot(hb, w2_ref[half:], preferred_element_type=jnp.float32))
    h2_ref[...] = h2
    hs2_ref[...] = h2 * dinv


def _final_body(s_ref, h2_ref, dinv_ref, b2_ref, o_ref):
    dinv = dinv_ref[...]
    s = s_ref[0] + s_ref[1]
    o = dinv * s + dinv * dinv * h2_ref[...] + b2_ref[...]
    m = jnp.max(o, axis=1, keepdims=True)
    z = o - m
    o_ref[...] = z - jnp.log(jnp.sum(jnp.exp(z), axis=1, keepdims=True))


def kernel(x, edge_index, W1, b1, W2, b2):
    n = x.shape[0]
    e = edge_index.shape[1]
    f_in = x.shape[1]
    hid = W1.shape[1]
    cls = W2.shape[1]
    half = hid // 2

    idxb1, idxb2 = 32, 16
    chunks16 = idxb1 * (-(-e // (NUM_SUBCORES * CHUNK * idxb1)))
    slots = NUM_SUBCORES * chunks16 * CHUNK
    assert slots % (NUM_TILES * CHUNK * idxb2) == 0
    pad = slots - e
    src_p = jnp.concatenate([edge_index[0], jnp.zeros((pad,), jnp.int32)])
    dst_p = jnp.concatenate([edge_index[1], jnp.full((pad,), n, jnp.int32)])
    src16 = src_p.reshape(NUM_SUBCORES, chunks16, CHUNK)
    dst16 = dst_p.reshape(NUM_SUBCORES, chunks16, CHUNK)
    src32 = src_p.reshape(NUM_TILES, chunks16 // 2, CHUNK)
    dst32 = dst_p.reshape(NUM_TILES, chunks16 // 2, CHUNK)

    n_acc = NUM_SUBCORES * CHUNK * (-(-(n + 1) // (NUM_SUBCORES * CHUNK)))

    hist = _make_sc_degree(n_acc, dst32.shape[1])(dst32)

    row_blk = 2000
    grid = (n // row_blk,)

    H1 = pl.pallas_call(
        _mm1_body,
        grid=grid,
        in_specs=[
            pl.BlockSpec((row_blk, f_in), lambda i: (i, 0)),
            pl.BlockSpec((f_in, hid), lambda i: (0, 0)),
        ],
        out_specs=pl.BlockSpec((row_blk, hid), lambda i: (i, 0)),
        out_shape=jax.ShapeDtypeStruct((n, hid), jnp.float32),
    )(x, W1)

    dinv, hs_a, hs_b = pl.pallas_call(
        _scale_body,
        grid=grid,
        in_specs=[
            pl.BlockSpec((NUM_CORES, row_blk, HIST_W), lambda i: (0, i, 0)),
            pl.BlockSpec((row_blk, hid), lambda i: (i, 0)),
        ],
        out_specs=[
            pl.BlockSpec((row_blk, 1), lambda i: (i, 0)),
            pl.BlockSpec((row_blk, half), lambda i: (i, 0)),
            pl.BlockSpec((row_blk, half), lambda i: (i, 0)),
        ],
        out_shape=[
            jax.ShapeDtypeStruct((n, 1), jnp.float32),
            jax.ShapeDtypeStruct((n, half), jnp.float32),
            jax.ShapeDtypeStruct((n, half), jnp.float32),
        ],
    )(hist, H1)

    S1 = _make_sc_gs(n_acc, n, half, True, idxb1)(hs_a, hs_b, src16, dst16)

    H2, Hs2 = pl.pallas_call(
        _mid_body,
        grid=grid,
        in_specs=[
            pl.BlockSpec((1, row_blk, half), lambda i: (0, i, 0)),
            pl.BlockSpec((1, row_blk, half), lambda i: (1, i, 0)),
            pl.BlockSpec((row_blk, hid), lambda i: (i, 0)),
            pl.BlockSpec((row_blk, 1), lambda i: (i, 0)),
            pl.BlockSpec((1, hid), lambda i: (0, 0)),
            pl.BlockSpec((hid, cls), lambda i: (0, 0)),
        ],
        out_specs=[
            pl.BlockSpec((row_blk, cls), lambda i: (i, 0)),
            pl.BlockSpec((row_blk, cls), lambda i: (i, 0)),
        ],
        out_shape=[jax.ShapeDtypeStruct((n, cls), jnp.float32)] * 2,
    )(S1, S1, H1, dinv, b1.reshape(1, hid), W2)

    S2 = _make_sc_gs(n_acc, n, cls, False, idxb2)(Hs2, src32, dst32)

    out = pl.pallas_call(
        _final_body,
        grid=grid,
        in_specs=[
            pl.BlockSpec((NUM_CORES, row_blk, cls), lambda i: (0, i, 0)),
            pl.BlockSpec((row_blk, cls), lambda i: (i, 0)),
            pl.BlockSpec((row_blk, 1), lambda i: (i, 0)),
            pl.BlockSpec((1, cls), lambda i: (0, 0)),
        ],
        out_specs=pl.BlockSpec((row_blk, cls), lambda i: (i, 0)),
        out_shape=jax.ShapeDtypeStruct((n, cls), jnp.float32),
    )(S2, H2, dinv, b2.reshape(1, cls))

    return out


# speedup vs baseline: 27.0386x; 1.0449x over previous
import functools

import jax
import jax.numpy as jnp
from jax import lax
from jax.experimental import pallas as pl
from jax.experimental.pallas import tpu as pltpu
from jax.experimental.pallas import tpu_sc as plsc

NUM_CORES = 2
NUM_SUBCORES = 16
NUM_TILES = NUM_CORES * NUM_SUBCORES
CHUNK = 128
NBUF = 4
HIST_W = 16
IDXB = 16


def _sc_degree_body(n_acc, dst_hbm, out_hbm, dst_v, ones_v, zero_v, acc):
    cid = lax.axis_index("c")
    sid = lax.axis_index("s")
    w = cid * NUM_SUBCORES + sid
    n_chunks = dst_v.shape[0]
    rows_per_tile = n_acc // NUM_SUBCORES

    def fill(i, carry):
        ones_v[i, :] = jnp.full((16,), 1.0, jnp.float32)
        zero_v[i, :] = jnp.zeros((16,), jnp.float32)
        return carry

    lax.fori_loop(0, CHUNK, fill, 0)

    def zero(j, carry):
        pltpu.sync_copy(
            zero_v, acc.at[pl.ds(sid * rows_per_tile + j * CHUNK, CHUNK)])
        return carry

    lax.fori_loop(0, rows_per_tile // CHUNK, zero, 0)
    pltpu.sync_copy(dst_hbm.at[w], dst_v)
    plsc.subcore_barrier()

    def body(j, carry):
        pltpu.sync_copy(ones_v, acc.at[dst_v.at[j]], add=True)
        return carry

    lax.fori_loop(0, n_chunks, body, 0)
    plsc.subcore_barrier()
    pltpu.sync_copy(
        acc.at[pl.ds(sid * rows_per_tile, rows_per_tile)],
        out_hbm.at[cid, pl.ds(sid * rows_per_tile, rows_per_tile)])


def _make_sc_degree(n_acc, n_chunks):
    mesh = plsc.VectorSubcoreMesh(core_axis_name="c", subcore_axis_name="s")
    return pl.kernel(
        functools.partial(_sc_degree_body, n_acc),
        out_type=jax.ShapeDtypeStruct((NUM_CORES, n_acc, HIST_W), jnp.float32),
        mesh=mesh,
        scratch_types=[
            pltpu.VMEM((n_chunks, CHUNK), jnp.int32),
            pltpu.VMEM((CHUNK, HIST_W), jnp.float32),
            pltpu.VMEM((CHUNK, HIST_W), jnp.float32),
            pltpu.VMEM_SHARED((n_acc, HIST_W), jnp.float32),
        ],
    )


def _sc_gs_body(n_acc, col_split, hs_hbm, src_hbm, dst_hbm, out_hbm, *rest):
    src_v, dst_v, rows = rest[:3]
    table, acc = rest[3:5]
    semg = rest[5:5 + NBUF]
    cid = lax.axis_index("c")
    sid = lax.axis_index("s")
    n_chunks = src_hbm.shape[1]
    feat = rows.shape[2]
    rows_per_tile = n_acc // NUM_SUBCORES
    stage_rows = table.shape[0] // NUM_SUBCORES
    col0 = cid * feat if col_split else 0

    pltpu.sync_copy(
        hs_hbm.at[pl.ds(sid * stage_rows, stage_rows), pl.ds(col0, feat)],
        table.at[pl.ds(sid * stage_rows, stage_rows)])

    def fill(i, carry):
        for c in range(feat // 16):
            rows[0, i, pl.ds(c * 16, 16)] = jnp.zeros((16,), jnp.float32)
        return carry

    lax.fori_loop(0, CHUNK, fill, 0)

    def zero(j, carry):
        pltpu.sync_copy(
            rows.at[0], acc.at[pl.ds(sid * rows_per_tile + j * CHUNK, CHUNK)])
        return carry

    lax.fori_loop(0, rows_per_tile // CHUNK, zero, 0)
    plsc.subcore_barrier()

    gpr = n_chunks // IDXB
    n_groups = 2 * gpr if col_split else gpr

    def group(g, carry):
        if col_split:
            row = 2 * sid + g // gpr
            gg = g % gpr
        else:
            row = cid * NUM_SUBCORES + sid
            gg = g
        pltpu.sync_copy(src_hbm.at[row, pl.ds(gg * IDXB, IDXB)], src_v)
        pltpu.sync_copy(dst_hbm.at[row, pl.ds(gg * IDXB, IDXB)], dst_v)
        for b in range(NBUF):
            pltpu.async_copy(table.at[src_v.at[b]], rows.at[b], semg[b])

        def body(t, carry2):
            for b in range(NBUF):
                j = NBUF * t + b
                pltpu.make_async_copy(
                    table.at[src_v.at[j]], rows.at[b], semg[b]).wait()
                pltpu.sync_copy(rows.at[b], acc.at[dst_v.at[j]], add=True)

                @pl.when(j + NBUF < IDXB)
                def _():
                    pltpu.async_copy(
                        table.at[src_v.at[j + NBUF]], rows.at[b], semg[b])

            return carry2

        lax.fori_loop(0, IDXB // NBUF, body, 0)
        return carry

    lax.fori_loop(0, n_groups, group, 0)
    plsc.subcore_barrier()
    pltpu.sync_copy(
        acc.at[pl.ds(sid * rows_per_tile, rows_per_tile)],
        out_hbm.at[pl.ds(sid * rows_per_tile, rows_per_tile),
                   pl.ds(cid * feat, feat)])


def _make_sc_gs(n_acc, n_table, feat, col_split):
    mesh = plsc.VectorSubcoreMesh(core_axis_name="c", subcore_axis_name="s")
    return pl.kernel(
        functools.partial(_sc_gs_body, n_acc, col_split),
        out_type=jax.ShapeDtypeStruct((n_acc, 2 * feat), jnp.float32),
        mesh=mesh,
        compiler_params=pltpu.CompilerParams(use_tc_tiling_on_sc=False),
        scratch_types=[
            pltpu.VMEM((IDXB, CHUNK), jnp.int32),
            pltpu.VMEM((IDXB, CHUNK), jnp.int32),
            pltpu.VMEM((NBUF, CHUNK, feat), jnp.float32),
            pltpu.VMEM_SHARED((n_table, feat), jnp.float32),
            pltpu.VMEM_SHARED((n_acc, feat), jnp.float32),
        ] + [pltpu.SemaphoreType.DMA] * NBUF,
    )


def _mm1_body(x_ref, w_ref, h_ref):
    h_ref[...] = jnp.dot(x_ref[...], w_ref[...],
                         preferred_element_type=jnp.float32)


def _scale_body(hist_ref, h1_ref, dinv_ref, hs_ref):
    deg = 1.0 + hist_ref[0, :, 0:1] + hist_ref[1, :, 0:1]
    dinv = lax.rsqrt(deg)
    dinv_ref[...] = dinv
    hs_ref[...] = h1_ref[...] * dinv


def _mid_body(s_ref, h1_ref, dinv_ref, b1_ref, w2_ref, mh_ref):
    dinv = dinv_ref[...]
    d2 = dinv * dinv
    h = jnp.maximum(dinv * s_ref[...] + d2 * h1_ref[...] + b1_ref[...], 0.0)
    h2 = jnp.dot(h, w2_ref[...], preferred_element_type=jnp.float32)
    mh_ref[...] = jnp.concatenate([h2 * dinv, h2], axis=1)


def _final_body(s_ref, mh_ref, dinv_ref, b2_ref, o_ref):
    dinv = dinv_ref[...]
    cls = o_ref.shape[1]
    s = s_ref[:, :cls] + s_ref[:, cls:]
    o = dinv * s + dinv * dinv * mh_ref[:, cls:] + b2_ref[...]
    m = jnp.max(o, axis=1, keepdims=True)
    z = o - m
    o_ref[...] = z - jnp.log(jnp.sum(jnp.exp(z), axis=1, keepdims=True))


def kernel(x, edge_index, W1, b1, W2, b2):
    n = x.shape[0]
    e = edge_index.shape[1]
    f_in = x.shape[1]
    hid = W1.shape[1]
    cls = W2.shape[1]
    half = hid // 2

    chunks32 = IDXB * (-(-e // (NUM_TILES * CHUNK * IDXB)))
    slots = NUM_TILES * chunks32 * CHUNK
    pad = slots - e
    src_p = jnp.concatenate([edge_index[0], jnp.zeros((pad,), jnp.int32)])
    dst_p = jnp.concatenate([edge_index[1], jnp.full((pad,), n, jnp.int32)])
    src32 = src_p.reshape(NUM_TILES, chunks32, CHUNK)
    dst32 = dst_p.reshape(NUM_TILES, chunks32, CHUNK)

    n_acc = NUM_SUBCORES * CHUNK * (-(-(n + 1) // (NUM_SUBCORES * CHUNK)))

    hist = _make_sc_degree(n_acc, chunks32)(dst32)

    row_blk = 2000
    grid = (n // row_blk,)

    H1 = pl.pallas_call(
        _mm1_body,
        grid=grid,
        in_specs=[
            pl.BlockSpec((row_blk, f_in), lambda i: (i, 0)),
            pl.BlockSpec((f_in, hid), lambda i: (0, 0)),
        ],
        out_specs=pl.BlockSpec((row_blk, hid), lambda i: (i, 0)),
        out_shape=jax.ShapeDtypeStruct((n, hid), jnp.float32),
    )(x, W1)

    dinv, hs = pl.pallas_call(
        _scale_body,
        grid=grid,
        in_specs=[
            pl.BlockSpec((NUM_CORES, row_blk, HIST_W), lambda i: (0, i, 0)),
            pl.BlockSpec((row_blk, hid), lambda i: (i, 0)),
        ],
        out_specs=[
            pl.BlockSpec((row_blk, 1), lambda i: (i, 0)),
            pl.BlockSpec((row_blk, hid), lambda i: (i, 0)),
        ],
        out_shape=[
            jax.ShapeDtypeStruct((n, 1), jnp.float32),
            jax.ShapeDtypeStruct((n, hid), jnp.float32),
        ],
    )(hist, H1)

    S1 = _make_sc_gs(n_acc, n, half, True)(hs, src32, dst32)

    MH = pl.pallas_call(
        _mid_body,
        grid=grid,
        in_specs=[
            pl.BlockSpec((row_blk, hid), lambda i: (i, 0)),
            pl.BlockSpec((row_blk, hid), lambda i: (i, 0)),
            pl.BlockSpec((row_blk, 1), lambda i: (i, 0)),
            pl.BlockSpec((1, hid), lambda i: (0, 0)),
            pl.BlockSpec((hid, cls), lambda i: (0, 0)),
        ],
        out_specs=pl.BlockSpec((row_blk, 2 * cls), lambda i: (i, 0)),
        out_shape=jax.ShapeDtypeStruct((n, 2 * cls), jnp.float32),
    )(S1, H1, dinv, b1.reshape(1, hid), W2)

    S2 = _make_sc_gs(n_acc, n, cls, False)(MH, src32, dst32)

    out = pl.pallas_call(
        _final_body,
        grid=grid,
        in_specs=[
            pl.BlockSpec((row_blk, 2 * cls), lambda i: (i, 0)),
            pl.BlockSpec((row_blk, 2 * cls), lambda i: (i, 0)),
            pl.BlockSpec((row_blk, 1), lambda i: (i, 0)),
            pl.BlockSpec((1, cls), lambda i: (0, 0)),
        ],
        out_specs=pl.BlockSpec((row_blk, cls), lambda i: (i, 0)),
        out_shape=jax.ShapeDtypeStruct((n, cls), jnp.float32),
    )(S2, MH, dinv, b2.reshape(1, cls))

    return out


# speedup vs baseline: 29.8597x; 1.1043x over previous
import functools

import jax
import jax.numpy as jnp
from jax import lax
from jax.experimental import pallas as pl
from jax.experimental.pallas import tpu as pltpu
from jax.experimental.pallas import tpu_sc as plsc

NUM_CORES = 2
NUM_SUBCORES = 16
NUM_TILES = NUM_CORES * NUM_SUBCORES
CHUNK = 128
NBUF = 4
HIST_W = 16
IDXB = 40


def _sc_degree_body(n_acc, dst_hbm, out_hbm, dst_v, ones_v, zero_v, acc):
    cid = lax.axis_index("c")
    sid = lax.axis_index("s")
    w = cid * NUM_SUBCORES + sid
    n_chunks = dst_v.shape[0]
    rows_per_tile = n_acc // NUM_SUBCORES

    def fill(i, carry):
        ones_v[i, :] = jnp.full((16,), 1.0, jnp.float32)
        zero_v[i, :] = jnp.zeros((16,), jnp.float32)
        return carry

    lax.fori_loop(0, CHUNK, fill, 0)

    def zero(j, carry):
        pltpu.sync_copy(
            zero_v, acc.at[pl.ds(sid * rows_per_tile + j * CHUNK, CHUNK)])
        return carry

    lax.fori_loop(0, rows_per_tile // CHUNK, zero, 0)
    pltpu.sync_copy(dst_hbm.at[w], dst_v)
    plsc.subcore_barrier()

    def body(j, carry):
        pltpu.sync_copy(ones_v, acc.at[dst_v.at[j]], add=True)
        return carry

    lax.fori_loop(0, n_chunks, body, 0)
    plsc.subcore_barrier()
    pltpu.sync_copy(
        acc.at[pl.ds(sid * rows_per_tile, rows_per_tile)],
        out_hbm.at[cid, pl.ds(sid * rows_per_tile, rows_per_tile)])


def _make_sc_degree(n_acc, n_chunks):
    mesh = plsc.VectorSubcoreMesh(core_axis_name="c", subcore_axis_name="s")
    return pl.kernel(
        functools.partial(_sc_degree_body, n_acc),
        out_type=jax.ShapeDtypeStruct((NUM_CORES, n_acc, HIST_W), jnp.float32),
        mesh=mesh,
        scratch_types=[
            pltpu.VMEM((n_chunks, CHUNK), jnp.int32),
            pltpu.VMEM((CHUNK, HIST_W), jnp.float32),
            pltpu.VMEM((CHUNK, HIST_W), jnp.float32),
            pltpu.VMEM_SHARED((n_acc, HIST_W), jnp.float32),
        ],
    )


def _sc_gs_body(n_acc, col_split, hs_hbm, src_hbm, dst_hbm, out_hbm, *rest):
    src_v, dst_v, rows = rest[:3]
    table, acc = rest[3:5]
    semg = rest[5:5 + NBUF]
    sem_stage = rest[5 + NBUF]
    cid = lax.axis_index("c")
    sid = lax.axis_index("s")
    n_chunks = src_hbm.shape[1]
    feat = rows.shape[2]
    rows_per_tile = n_acc // NUM_SUBCORES
    stage_rows = table.shape[0] // NUM_SUBCORES
    col0 = cid * feat if col_split else 0

    pltpu.async_copy(
        hs_hbm.at[pl.ds(sid * stage_rows, stage_rows), pl.ds(col0, feat)],
        table.at[pl.ds(sid * stage_rows, stage_rows)], sem_stage)

    def fill(i, carry):
        for c in range(feat // 16):
            rows[0, i, pl.ds(c * 16, 16)] = jnp.zeros((16,), jnp.float32)
        return carry

    lax.fori_loop(0, CHUNK, fill, 0)

    def zero(j, carry):
        pltpu.sync_copy(
            rows.at[0], acc.at[pl.ds(sid * rows_per_tile + j * CHUNK, CHUNK)])
        return carry

    lax.fori_loop(0, rows_per_tile // CHUNK, zero, 0)
    pltpu.make_async_copy(
        hs_hbm.at[pl.ds(sid * stage_rows, stage_rows), pl.ds(col0, feat)],
        table.at[pl.ds(sid * stage_rows, stage_rows)], sem_stage).wait()
    plsc.subcore_barrier()

    gpr = n_chunks // IDXB
    n_groups = 2 * gpr if col_split else gpr

    def group(g, carry):
        if col_split:
            row = 2 * sid + g // gpr
            gg = g % gpr
        else:
            row = cid * NUM_SUBCORES + sid
            gg = g
        pltpu.sync_copy(src_hbm.at[row, pl.ds(gg * IDXB, IDXB)], src_v)
        pltpu.sync_copy(dst_hbm.at[row, pl.ds(gg * IDXB, IDXB)], dst_v)
        for b in range(NBUF):
            pltpu.async_copy(table.at[src_v.at[b]], rows.at[b], semg[b])

        def body(t, carry2):
            for b in range(NBUF):
                j = NBUF * t + b
                pltpu.make_async_copy(
                    table.at[src_v.at[j]], rows.at[b], semg[b]).wait()
                pltpu.sync_copy(rows.at[b], acc.at[dst_v.at[j]], add=True)

                @pl.when(j + NBUF < IDXB)
                def _():
                    pltpu.async_copy(
                        table.at[src_v.at[j + NBUF]], rows.at[b], semg[b])

            return carry2

        lax.fori_loop(0, IDXB // NBUF, body, 0)
        return carry

    lax.fori_loop(0, n_groups, group, 0)
    plsc.subcore_barrier()
    pltpu.sync_copy(
        acc.at[pl.ds(sid * rows_per_tile, rows_per_tile)],
        out_hbm.at[pl.ds(sid * rows_per_tile, rows_per_tile),
                   pl.ds(cid * feat, feat)])


def _make_sc_gs(n_acc, n_table, feat, col_split):
    mesh = plsc.VectorSubcoreMesh(core_axis_name="c", subcore_axis_name="s")
    return pl.kernel(
        functools.partial(_sc_gs_body, n_acc, col_split),
        out_type=jax.ShapeDtypeStruct((n_acc, 2 * feat), jnp.float32),
        mesh=mesh,
        compiler_params=pltpu.CompilerParams(use_tc_tiling_on_sc=False),
        scratch_types=[
            pltpu.VMEM((IDXB, CHUNK), jnp.int32),
            pltpu.VMEM((IDXB, CHUNK), jnp.int32),
            pltpu.VMEM((NBUF, CHUNK, feat), jnp.float32),
            pltpu.VMEM_SHARED((n_table, feat), jnp.float32),
            pltpu.VMEM_SHARED((n_acc, feat), jnp.float32),
        ] + [pltpu.SemaphoreType.DMA] * (NBUF + 1),
    )


def _mm1_body(x_ref, w_ref, h_ref):
    h_ref[...] = jnp.dot(x_ref[...], w_ref[...],
                         preferred_element_type=jnp.float32)


def _scale_body(hist_ref, h1_ref, dinv_ref, hs_ref):
    deg = 1.0 + hist_ref[0, :, 0:1] + hist_ref[1, :, 0:1]
    dinv = lax.rsqrt(deg)
    dinv_ref[...] = dinv
    hs_ref[...] = h1_ref[...] * dinv


def _mid_body(s_ref, h1_ref, dinv_ref, b1_ref, w2_ref, mh_ref):
    dinv = dinv_ref[...]
    d2 = dinv * dinv
    h = jnp.maximum(dinv * s_ref[...] + d2 * h1_ref[...] + b1_ref[...], 0.0)
    h2 = jnp.dot(h, w2_ref[...], preferred_element_type=jnp.float32)
    mh_ref[...] = jnp.concatenate([h2 * dinv, h2], axis=1)


def _final_body(s_ref, mh_ref, dinv_ref, b2_ref, o_ref):
    dinv = dinv_ref[...]
    cls = o_ref.shape[1]
    s = s_ref[:, :cls] + s_ref[:, cls:]
    o = dinv * s + dinv * dinv * mh_ref[:, cls:] + b2_ref[...]
    m = jnp.max(o, axis=1, keepdims=True)
    z = o - m
    o_ref[...] = z - jnp.log(jnp.sum(jnp.exp(z), axis=1, keepdims=True))


def kernel(x, edge_index, W1, b1, W2, b2):
    n = x.shape[0]
    e = edge_index.shape[1]
    f_in = x.shape[1]
    hid = W1.shape[1]
    cls = W2.shape[1]
    half = hid // 2

    chunks32 = IDXB * (-(-e // (NUM_TILES * CHUNK * IDXB)))
    slots = NUM_TILES * chunks32 * CHUNK
    pad = slots - e
    src_p = jnp.concatenate([edge_index[0], jnp.zeros((pad,), jnp.int32)])
    dst_p = jnp.concatenate([edge_index[1], jnp.full((pad,), n, jnp.int32)])
    src32 = src_p.reshape(NUM_TILES, chunks32, CHUNK)
    dst32 = dst_p.reshape(NUM_TILES, chunks32, CHUNK)

    n_acc = NUM_SUBCORES * CHUNK * (-(-(n + 1) // (NUM_SUBCORES * CHUNK)))

    hist = _make_sc_degree(n_acc, chunks32)(dst32)

    row_blk = 2000
    grid = (n // row_blk,)

    H1 = pl.pallas_call(
        _mm1_body,
        grid=grid,
        in_specs=[
            pl.BlockSpec((row_blk, f_in), lambda i: (i, 0)),
            pl.BlockSpec((f_in, hid), lambda i: (0, 0)),
        ],
        out_specs=pl.BlockSpec((row_blk, hid), lambda i: (i, 0)),
        out_shape=jax.ShapeDtypeStruct((n, hid), jnp.float32),
    )(x, W1)

    dinv, hs = pl.pallas_call(
        _scale_body,
        grid=grid,
        in_specs=[
            pl.BlockSpec((NUM_CORES, row_blk, HIST_W), lambda i: (0, i, 0)),
            pl.BlockSpec((row_blk, hid), lambda i: (i, 0)),
        ],
        out_specs=[
            pl.BlockSpec((row_blk, 1), lambda i: (i, 0)),
            pl.BlockSpec((row_blk, hid), lambda i: (i, 0)),
        ],
        out_shape=[
            jax.ShapeDtypeStruct((n, 1), jnp.float32),
            jax.ShapeDtypeStruct((n, hid), jnp.float32),
        ],
    )(hist, H1)

    S1 = _make_sc_gs(n_acc, n, half, True)(hs, src32, dst32)

    MH = pl.pallas_call(
        _mid_body,
        grid=grid,
        in_specs=[
            pl.BlockSpec((row_blk, hid), lambda i: (i, 0)),
            pl.BlockSpec((row_blk, hid), lambda i: (i, 0)),
            pl.BlockSpec((row_blk, 1), lambda i: (i, 0)),
            pl.BlockSpec((1, hid), lambda i: (0, 0)),
            pl.BlockSpec((hid, cls), lambda i: (0, 0)),
        ],
        out_specs=pl.BlockSpec((row_blk, 2 * cls), lambda i: (i, 0)),
        out_shape=jax.ShapeDtypeStruct((n, 2 * cls), jnp.float32),
    )(S1, H1, dinv, b1.reshape(1, hid), W2)

    S2 = _make_sc_gs(n_acc, n, cls, False)(MH, src32, dst32)

    out = pl.pallas_call(
        _final_body,
        grid=grid,
        in_specs=[
            pl.BlockSpec((row_blk, 2 * cls), lambda i: (i, 0)),
            pl.BlockSpec((row_blk, 2 * cls), lambda i: (i, 0)),
            pl.BlockSpec((row_blk, 1), lambda i: (i, 0)),
            pl.BlockSpec((1, cls), lambda i: (0, 0)),
        ],
        out_specs=pl.BlockSpec((row_blk, cls), lambda i: (i, 0)),
        out_shape=jax.ShapeDtypeStruct((n, cls), jnp.float32),
    )(S2, MH, dinv, b2.reshape(1, cls))

    return out


# speedup vs baseline: 31.3084x; 1.0485x over previous
import functools

import jax
import jax.numpy as jnp
from jax import lax
from jax.experimental import pallas as pl
from jax.experimental.pallas import tpu as pltpu
from jax.experimental.pallas import tpu_sc as plsc

NUM_CORES = 2
NUM_SUBCORES = 16
NUM_TILES = NUM_CORES * NUM_SUBCORES
CHUNK = 128
NBUF = 4
HIST_W = 16
IDXB = 39


def _sc_degree_body(n_acc, n_chunks, eidx_hbm, out_hbm, dst_v, ones_v,
                    zero_v, acc):
    cid = lax.axis_index("c")
    sid = lax.axis_index("s")
    w = cid * NUM_SUBCORES + sid
    base = n_chunks // NUM_TILES
    rem = n_chunks % NUM_TILES
    off = base * w + jnp.minimum(w, rem)
    rows_per_tile = n_acc // NUM_SUBCORES

    def fill(i, carry):
        ones_v[i, :] = jnp.full((16,), 1.0, jnp.float32)
        zero_v[i, :] = jnp.zeros((16,), jnp.float32)
        return carry

    lax.fori_loop(0, CHUNK, fill, 0)

    def zero(j, carry):
        pltpu.sync_copy(
            zero_v, acc.at[pl.ds(sid * rows_per_tile + j * CHUNK, CHUNK)])
        return carry

    lax.fori_loop(0, rows_per_tile // CHUNK, zero, 0)
    pltpu.sync_copy(eidx_hbm.at[1, pl.ds(off * CHUNK, base * CHUNK)],
                    dst_v.at[pl.ds(0, base * CHUNK)])
    if rem:
        @pl.when(w < rem)
        def _():
            pltpu.sync_copy(
                eidx_hbm.at[1, pl.ds((off + base) * CHUNK, CHUNK)],
                dst_v.at[pl.ds(base * CHUNK, CHUNK)])

    plsc.subcore_barrier()

    def body(j, carry):
        pltpu.sync_copy(
            ones_v, acc.at[dst_v.at[pl.ds(j * CHUNK, CHUNK)]], add=True)
        return carry

    lax.fori_loop(0, base, body, 0)
    if rem:
        @pl.when(w < rem)
        def _():
            pltpu.sync_copy(
                ones_v, acc.at[dst_v.at[pl.ds(base * CHUNK, CHUNK)]],
                add=True)

    plsc.subcore_barrier()
    pltpu.sync_copy(
        acc.at[pl.ds(sid * rows_per_tile, rows_per_tile)],
        out_hbm.at[cid, pl.ds(sid * rows_per_tile, rows_per_tile)])


def _make_sc_degree(n_acc, n_chunks):
    mesh = plsc.VectorSubcoreMesh(core_axis_name="c", subcore_axis_name="s")
    max_c = n_chunks // NUM_TILES + (1 if n_chunks % NUM_TILES else 0)
    return pl.kernel(
        functools.partial(_sc_degree_body, n_acc, n_chunks),
        out_type=jax.ShapeDtypeStruct((NUM_CORES, n_acc, HIST_W), jnp.float32),
        mesh=mesh,
        compiler_params=pltpu.CompilerParams(use_tc_tiling_on_sc=False),
        scratch_types=[
            pltpu.VMEM((max_c * CHUNK,), jnp.int32),
            pltpu.VMEM((CHUNK, HIST_W), jnp.float32),
            pltpu.VMEM((CHUNK, HIST_W), jnp.float32),
            pltpu.VMEM_SHARED((n_acc, HIST_W), jnp.float32),
        ],
    )


def _sc_gs_body(n_acc, col_split, n_chunks, hs_hbm, eidx_hbm, out_hbm, *rest):
    src_v, dst_v, rows = rest[:3]
    table, acc = rest[3:5]
    semg = rest[5:5 + NBUF]
    sem_stage = rest[5 + NBUF]
    cid = lax.axis_index("c")
    sid = lax.axis_index("s")
    feat = rows.shape[2]
    rows_per_tile = n_acc // NUM_SUBCORES
    stage_rows = table.shape[0] // NUM_SUBCORES
    col0 = cid * feat if col_split else 0
    n_own = NUM_SUBCORES if col_split else NUM_TILES
    w = sid if col_split else cid * NUM_SUBCORES + sid
    base = n_chunks // n_own
    rem = n_chunks % n_own
    off = base * w + jnp.minimum(w, rem)

    pltpu.async_copy(
        hs_hbm.at[pl.ds(sid * stage_rows, stage_rows), pl.ds(col0, feat)],
        table.at[pl.ds(sid * stage_rows, stage_rows)], sem_stage)

    def fill(i, carry):
        for c in range(feat // 16):
            rows[0, i, pl.ds(c * 16, 16)] = jnp.zeros((16,), jnp.float32)
        return carry

    lax.fori_loop(0, CHUNK, fill, 0)

    def zero(j, carry):
        pltpu.sync_copy(
            rows.at[0], acc.at[pl.ds(sid * rows_per_tile + j * CHUNK, CHUNK)])
        return carry

    lax.fori_loop(0, rows_per_tile // CHUNK, zero, 0)
    pltpu.make_async_copy(
        hs_hbm.at[pl.ds(sid * stage_rows, stage_rows), pl.ds(col0, feat)],
        table.at[pl.ds(sid * stage_rows, stage_rows)], sem_stage).wait()
    plsc.subcore_barrier()

    def run_seg(seg_len, chunk0):
        if seg_len == 0:
            return
        pltpu.sync_copy(eidx_hbm.at[0, pl.ds(chunk0 * CHUNK, seg_len * CHUNK)],
                        src_v.at[pl.ds(0, seg_len * CHUNK)])
        pltpu.sync_copy(eidx_hbm.at[1, pl.ds(chunk0 * CHUNK, seg_len * CHUNK)],
                        dst_v.at[pl.ds(0, seg_len * CHUNK)])
        for b in range(min(NBUF, seg_len)):
            pltpu.async_copy(
                table.at[src_v.at[pl.ds(b * CHUNK, CHUNK)]], rows.at[b],
                semg[b])

        def step(j, b):
            pltpu.make_async_copy(
                table.at[src_v.at[pl.ds(j * CHUNK, CHUNK)]], rows.at[b],
                semg[b]).wait()
            pltpu.sync_copy(
                rows.at[b], acc.at[dst_v.at[pl.ds(j * CHUNK, CHUNK)]],
                add=True)

            @pl.when(j + NBUF < seg_len)
            def _():
                pltpu.async_copy(
                    table.at[src_v.at[pl.ds((j + NBUF) * CHUNK, CHUNK)]],
                    rows.at[b], semg[b])

        if seg_len // NBUF:
            def body(t, carry):
                for b in range(NBUF):
                    step(NBUF * t + b, b)
                return carry

            lax.fori_loop(0, seg_len // NBUF, body, 0)
        for b in range(seg_len % NBUF):
            step((seg_len // NBUF) * NBUF + b, b)

    def group(g, carry):
        run_seg(IDXB, off + g * IDXB)
        return carry

    if base // IDXB:
        lax.fori_loop(0, base // IDXB, group, 0)
    run_seg(base % IDXB, off + (base // IDXB) * IDXB)
    if rem:
        @pl.when(w < rem)
        def _():
            run_seg(1, off + base)

    plsc.subcore_barrier()
    pltpu.sync_copy(
        acc.at[pl.ds(sid * rows_per_tile, rows_per_tile)],
        out_hbm.at[pl.ds(sid * rows_per_tile, rows_per_tile),
                   pl.ds(cid * feat, feat)])


def _make_sc_gs(n_acc, n_table, feat, col_split, n_chunks):
    mesh = plsc.VectorSubcoreMesh(core_axis_name="c", subcore_axis_name="s")
    return pl.kernel(
        functools.partial(_sc_gs_body, n_acc, col_split, n_chunks),
        out_type=jax.ShapeDtypeStruct((n_acc, 2 * feat), jnp.float32),
        mesh=mesh,
        compiler_params=pltpu.CompilerParams(use_tc_tiling_on_sc=False),
        scratch_types=[
            pltpu.VMEM((IDXB * CHUNK,), jnp.int32),
            pltpu.VMEM((IDXB * CHUNK,), jnp.int32),
            pltpu.VMEM((NBUF, CHUNK, feat), jnp.float32),
            pltpu.VMEM_SHARED((n_table, feat), jnp.float32),
            pltpu.VMEM_SHARED((n_acc, feat), jnp.float32),
        ] + [pltpu.SemaphoreType.DMA] * (NBUF + 1),
    )


def _mm1_body(x_ref, w_ref, h_ref):
    h_ref[...] = jnp.dot(x_ref[...], w_ref[...],
                         preferred_element_type=jnp.float32)


def _scale_body(hist_ref, h1_ref, dinv_ref, hs_ref):
    deg = 1.0 + hist_ref[0, :, 0:1] + hist_ref[1, :, 0:1]
    dinv = lax.rsqrt(deg)
    dinv_ref[...] = dinv
    hs_ref[...] = h1_ref[...] * dinv


def _mid_body(s_ref, h1_ref, dinv_ref, b1_ref, w2_ref, mh_ref):
    dinv = dinv_ref[...]
    d2 = dinv * dinv
    h = jnp.maximum(dinv * s_ref[...] + d2 * h1_ref[...] + b1_ref[...], 0.0)
    h2 = jnp.dot(h, w2_ref[...], preferred_element_type=jnp.float32)
    mh_ref[...] = jnp.concatenate([h2 * dinv, h2], axis=1)


def _final_body(s_ref, mh_ref, dinv_ref, b2_ref, o_ref):
    dinv = dinv_ref[...]
    cls = o_ref.shape[1]
    s = s_ref[:, :cls] + s_ref[:, cls:]
    o = dinv * s + dinv * dinv * mh_ref[:, cls:] + b2_ref[...]
    m = jnp.max(o, axis=1, keepdims=True)
    z = o - m
    o_ref[...] = z - jnp.log(jnp.sum(jnp.exp(z), axis=1, keepdims=True))


def kernel(x, edge_index, W1, b1, W2, b2):
    n = x.shape[0]
    e = edge_index.shape[1]
    f_in = x.shape[1]
    hid = W1.shape[1]
    cls = W2.shape[1]
    half = hid // 2

    if e % CHUNK:
        pad = CHUNK - e % CHUNK
        eidx = jnp.concatenate(
            [edge_index,
             jnp.stack([jnp.zeros((pad,), jnp.int32),
                        jnp.full((pad,), n, jnp.int32)])], axis=1)
    else:
        eidx = edge_index
    n_chunks = eidx.shape[1] // CHUNK

    n_acc = NUM_SUBCORES * CHUNK * (-(-(n + 1) // (NUM_SUBCORES * CHUNK)))

    hist = _make_sc_degree(n_acc, n_chunks)(eidx)

    row_blk = 2000
    grid = (n // row_blk,)

    H1 = pl.pallas_call(
        _mm1_body,
        grid=grid,
        in_specs=[
            pl.BlockSpec((row_blk, f_in), lambda i: (i, 0)),
            pl.BlockSpec((f_in, hid), lambda i: (0, 0)),
        ],
        out_specs=pl.BlockSpec((row_blk, hid), lambda i: (i, 0)),
        out_shape=jax.ShapeDtypeStruct((n, hid), jnp.float32),
    )(x, W1)

    dinv, hs = pl.pallas_call(
        _scale_body,
        grid=grid,
        in_specs=[
            pl.BlockSpec((NUM_CORES, row_blk, HIST_W), lambda i: (0, i, 0)),
            pl.BlockSpec((row_blk, hid), lambda i: (i, 0)),
        ],
        out_specs=[
            pl.BlockSpec((row_blk, 1), lambda i: (i, 0)),
            pl.BlockSpec((row_blk, hid), lambda i: (i, 0)),
        ],
        out_shape=[
            jax.ShapeDtypeStruct((n, 1), jnp.float32),
            jax.ShapeDtypeStruct((n, hid), jnp.float32),
        ],
    )(hist, H1)

    S1 = _make_sc_gs(n_acc, n, half, True, n_chunks)(hs, eidx)

    MH = pl.pallas_call(
        _mid_body,
        grid=grid,
        in_specs=[
            pl.BlockSpec((row_blk, hid), lambda i: (i, 0)),
            pl.BlockSpec((row_blk, hid), lambda i: (i, 0)),
            pl.BlockSpec((row_blk, 1), lambda i: (i, 0)),
            pl.BlockSpec((1, hid), lambda i: (0, 0)),
            pl.BlockSpec((hid, cls), lambda i: (0, 0)),
        ],
        out_specs=pl.BlockSpec((row_blk, 2 * cls), lambda i: (i, 0)),
        out_shape=jax.ShapeDtypeStruct((n, 2 * cls), jnp.float32),
    )(S1, H1, dinv, b1.reshape(1, hid), W2)

    S2 = _make_sc_gs(n_acc, n, cls, False, n_chunks)(MH, eidx)

    out = pl.pallas_call(
        _final_body,
        grid=grid,
        in_specs=[
            pl.BlockSpec((row_blk, 2 * cls), lambda i: (i, 0)),
            pl.BlockSpec((row_blk, 2 * cls), lambda i: (i, 0)),
            pl.BlockSpec((row_blk, 1), lambda i: (i, 0)),
            pl.BlockSpec((1, cls), lambda i: (0, 0)),
        ],
        out_specs=pl.BlockSpec((row_blk, cls), lambda i: (i, 0)),
        out_shape=jax.ShapeDtypeStruct((n, cls), jnp.float32),
    )(S2, MH, dinv, b2.reshape(1, cls))

    return out
